# baseline (speedup 1.0000x reference)
import functools

import jax
import jax.numpy as jnp
from jax import lax
from jax.experimental import pallas as pl
from jax.experimental.pallas import tpu as pltpu
from jax.experimental.pallas import tpu_sc as plsc

NUM_WORKERS = 32
CHUNK = 128


@functools.partial(jax.jit, static_argnames=("b_total", "dim"))
def _sc_embed(idx_flat, table, *, b_total, dim):
    b_per_w = b_total // NUM_WORKERS
    n_chunks = b_per_w // CHUNK

    mesh = plsc.VectorSubcoreMesh(core_axis_name="c", subcore_axis_name="s")

    @functools.partial(
        pl.kernel,
        out_type=jax.ShapeDtypeStruct((b_total, dim), jnp.float32),
        mesh=mesh,
        scratch_types=[
            pltpu.VMEM((CHUNK,), jnp.int32),
            pltpu.VMEM((CHUNK, dim), jnp.float32),
            pltpu.SemaphoreType.DMA,
        ],
        compiler_params=pltpu.CompilerParams(use_tc_tiling_on_sc=False),
    )
    def k(idx_hbm, table_hbm, out_hbm, idx_v, rows_v, sem):
        wid = lax.axis_index("s") * 2 + lax.axis_index("c")
        base = wid * b_per_w

        def body(i, carry):
            off = base + i * CHUNK
            pltpu.sync_copy(idx_hbm.at[pl.ds(off, CHUNK)], idx_v)
            pltpu.async_copy(table_hbm.at[idx_v], rows_v, sem).wait()
            pltpu.sync_copy(rows_v, out_hbm.at[pl.ds(off, CHUNK)])
            return carry

        lax.fori_loop(0, n_chunks, body, 0)

    return k(idx_flat, table)


def kernel(input_ids, token_embed):
    batch, seq = input_ids.shape
    dim = token_embed.shape[1]
    idx_flat = input_ids.reshape(-1)
    out = _sc_embed(idx_flat, token_embed, b_total=batch * seq, dim=dim)
    return out.reshape(batch, seq, dim)


# speedup vs baseline: 1.1949x; 1.1949x over previous
import functools

import jax
import jax.numpy as jnp
from jax import lax
from jax.experimental import pallas as pl
from jax.experimental.pallas import tpu as pltpu
from jax.experimental.pallas import tpu_sc as plsc

NUM_WORKERS = 32
CHUNK = 128
K = 8


@functools.partial(jax.jit, static_argnames=("b_total", "dim"))
def _sc_embed(idx_3d, table, *, b_total, dim):
    b_per_w = b_total // NUM_WORKERS
    n_chunks = b_per_w // CHUNK
    n_groups = n_chunks // K

    mesh = plsc.VectorSubcoreMesh(core_axis_name="c", subcore_axis_name="s")

    @functools.partial(
        pl.kernel,
        out_type=jax.ShapeDtypeStruct((b_total, dim), jnp.float32),
        mesh=mesh,
        scratch_types=[
            pltpu.VMEM((n_chunks, CHUNK), jnp.int32),
            pltpu.VMEM((K, CHUNK, dim), jnp.float32),
            pltpu.SemaphoreType.DMA((K,)),
            pltpu.SemaphoreType.DMA((K,)),
        ],
        compiler_params=pltpu.CompilerParams(use_tc_tiling_on_sc=False),
    )
    def k(idx_hbm, table_hbm, out_hbm, idx_v, rows_v, gsem, ssem):
        wid = lax.axis_index("s") * 2 + lax.axis_index("c")
        base = wid * b_per_w
        pltpu.sync_copy(idx_hbm.at[wid], idx_v)

        def body(g, carry):
            c0 = g * K
            for b in range(K):
                pltpu.async_copy(
                    table_hbm.at[idx_v.at[c0 + b]], rows_v.at[b], gsem.at[b]
                )
            for b in range(K):
                off = base + (c0 + b) * CHUNK
                pltpu.make_async_copy(
                    table_hbm.at[idx_v.at[c0 + b]], rows_v.at[b], gsem.at[b]
                ).wait()
                pltpu.async_copy(
                    rows_v.at[b], out_hbm.at[pl.ds(off, CHUNK)], ssem.at[b]
                )
            for b in range(K):
                off = base + (c0 + b) * CHUNK
                pltpu.make_async_copy(
                    rows_v.at[b], out_hbm.at[pl.ds(off, CHUNK)], ssem.at[b]
                ).wait()
            return carry

        lax.fori_loop(0, n_groups, body, 0)

    return k(idx_3d, table)


def kernel(input_ids, token_embed):
    batch, seq = input_ids.shape
    dim = token_embed.shape[1]
    b_total = batch * seq
    b_per_w = b_total // NUM_WORKERS
    idx_3d = input_ids.reshape(NUM_WORKERS, b_per_w // CHUNK, CHUNK)
    out = _sc_embed(idx_3d, token_embed, b_total=b_total, dim=dim)
    return out.reshape(batch, seq, dim)
